# Initial kernel scaffold; baseline (speedup 1.0000x reference)
import jax
import jax.numpy as jnp
from jax.experimental import pallas as pl
from jax.experimental.pallas import tpu as pltpu


def kernel(atom_fea, nbr_fea, nbr_fea_idx, crystal_atom_idx, W_atom, b_atom, W_edge, b_edge, W_e1, b_e1, W_e2, b_e2, W_n1, b_n1, W_n2, b_n2, W_r, b_r, W_o, b_o):
    N = atom_fea.shape[0]
    M = nbr_fea.shape[1]
    BN = 250
    BE = BN * M

    def body(node_ref, edge_ref, w_ref, out_ref):
        ps = node_ref[...][:, :64]
        ps3 = jnp.broadcast_to(ps[:, None, :], (BN, M, 64))
        psr = ps3.reshape(BE, 64)
        e = edge_ref[...]
        pre = jax.lax.dot_general(e, w_ref[...], (((1,), (0,)), ((), ())),
                                  preferred_element_type=jnp.float32,
                                  precision=jax.lax.Precision.HIGHEST)
        h = pre + psr
        h = h * jax.nn.sigmoid(h)
        out_ref[...] = h

    edge_attr = nbr_fea.reshape(N * M, 16)
    out = pl.pallas_call(
        body,
        grid=(N // BN,),
        in_specs=[
            pl.BlockSpec((BN, 128), lambda i: (i, 0)),
            pl.BlockSpec((BE, 16), lambda i: (i, 0)),
            pl.BlockSpec((16, 64), lambda i: (0, 0)),
        ],
        out_specs=pl.BlockSpec((BE, 64), lambda i: (i, 0)),
        out_shape=jax.ShapeDtypeStruct((N * M, 64), jnp.float32),
    )(atom_fea, edge_attr, W_e1[0, :16, :])
    return jnp.sum(out[:100], axis=1, keepdims=True)


# speedup vs baseline: 2.4625x; 2.4625x over previous
import functools

import jax
import jax.numpy as jnp
from jax import lax
from jax.experimental import pallas as pl
from jax.experimental.pallas import tpu as pltpu
from jax.experimental.pallas import tpu_sc as plsc

_F32 = jnp.float32
_PREC = jax.lax.Precision.HIGHEST


def _silu(x):
    return x * jax.nn.sigmoid(x)


def _sc_gather(table, idx, chunk):
    n_rows, d = table.shape
    e = idx.shape[0]
    nw = 32
    epw = e // nw
    nch = epw // chunk
    mesh = plsc.VectorSubcoreMesh(core_axis_name="c", subcore_axis_name="s")

    @functools.partial(
        pl.kernel,
        mesh=mesh,
        compiler_params=pltpu.CompilerParams(use_tc_tiling_on_sc=False, needs_layout_passes=False),
        out_type=jax.ShapeDtypeStruct((e, d), _F32),
        scratch_types=[
            pltpu.VMEM((chunk,), jnp.int32),
            pltpu.VMEM((chunk, d), _F32),
            pltpu.SemaphoreType.DMA,
        ],
    )
    def k(table_hbm, idx_hbm, out_hbm, idx_v, rows_v, sem):
        wid = lax.axis_index("s") * 2 + lax.axis_index("c")

        def body(c, carry):
            base = wid * epw + c * chunk
            pltpu.sync_copy(idx_hbm.at[pl.ds(base, chunk)], idx_v)
            pltpu.async_copy(table_hbm.at[idx_v], rows_v, sem).wait()
            pltpu.sync_copy(rows_v, out_hbm.at[pl.ds(base, chunk)])
            return carry

        lax.fori_loop(0, nch, body, 0)

    return k(table, idx)


def _sc_scatter_add(vals, idx, zinit, chunk):
    e = vals.shape[0]
    n_rows = zinit.shape[0]
    nw = 32
    epw = e // nw
    nch = epw // chunk
    rps = n_rows // 16
    mesh = plsc.VectorSubcoreMesh(core_axis_name="c", subcore_axis_name="s")

    @functools.partial(
        pl.kernel,
        mesh=mesh,
        compiler_params=pltpu.CompilerParams(use_tc_tiling_on_sc=False, needs_layout_passes=False),
        out_type=jax.ShapeDtypeStruct((2, n_rows, 16), _F32),
        scratch_types=[
            pltpu.VMEM((chunk,), jnp.int32),
            pltpu.VMEM((chunk, 16), _F32),
            pltpu.VMEM_SHARED((n_rows, 16), _F32),
            pltpu.SemaphoreType.DMA,
        ],
    )
    def k(vals_hbm, idx_hbm, zinit_hbm, out_hbm, idx_v, vals_v, shared, sem):
        cid = lax.axis_index("c")
        sid = lax.axis_index("s")
        wid = sid * 2 + cid
        pltpu.sync_copy(zinit_hbm.at[pl.ds(sid * rps, rps)],
                        vals_v.at[pl.ds(0, rps)])
        pltpu.sync_copy(vals_v.at[pl.ds(0, rps)],
                        shared.at[pl.ds(sid * rps, rps)])
        plsc.subcore_barrier()

        def body(c, carry):
            base = wid * epw + c * chunk
            pltpu.sync_copy(idx_hbm.at[pl.ds(base, chunk)], idx_v)
            pltpu.sync_copy(vals_hbm.at[pl.ds(base, chunk)], vals_v)
            pltpu.sync_copy(vals_v, shared.at[idx_v], add=True)
            return carry

        lax.fori_loop(0, nch, body, 0)
        plsc.subcore_barrier()
        pltpu.sync_copy(shared.at[pl.ds(sid * rps, rps)],
                        vals_v.at[pl.ds(0, rps)])
        pltpu.sync_copy(vals_v.at[pl.ds(0, rps)],
                        out_hbm.at[cid, pl.ds(sid * rps, rps)])

    return k(vals, idx, zinit)


def _sc_counts(maskf, idx, zeros1d, chunk):
    e = maskf.shape[0]
    n_rows = zeros1d.shape[0]
    nw = 32
    epw = e // nw
    nch = epw // chunk
    mesh = plsc.VectorSubcoreMesh(core_axis_name="c", subcore_axis_name="s")

    @functools.partial(
        pl.kernel,
        mesh=mesh,
        compiler_params=pltpu.CompilerParams(use_tc_tiling_on_sc=False, needs_layout_passes=False),
        out_type=jax.ShapeDtypeStruct((nw, n_rows), _F32),
        scratch_types=[
            pltpu.VMEM((chunk,), jnp.int32),
            pltpu.VMEM((chunk,), _F32),
            pltpu.VMEM((n_rows,), _F32),
        ],
    )
    def k(mask_hbm, idx_hbm, z_hbm, out_hbm, idx_v, mask_v, cnt_v):
        wid = lax.axis_index("s") * 2 + lax.axis_index("c")
        pltpu.sync_copy(z_hbm, cnt_v)

        def chunk_body(c, carry):
            base = wid * epw + c * chunk
            pltpu.sync_copy(idx_hbm.at[pl.ds(base, chunk)], idx_v)
            pltpu.sync_copy(mask_hbm.at[pl.ds(base, chunk)], mask_v)

            def vec_body(j, carry2):
                i16 = idx_v[pl.ds(j * 16, 16)]
                m16 = mask_v[pl.ds(j * 16, 16)]
                plsc.addupdate_scatter(cnt_v, [i16], m16)
                return carry2

            lax.fori_loop(0, chunk // 16, vec_body, 0)
            return carry

        lax.fori_loop(0, nch, chunk_body, 0)
        pltpu.sync_copy(cnt_v, out_hbm.at[wid])

    return k(maskf, idx, zeros1d)


def _tc_init(atom_fea, edge_attr, W_atom, b_atom, W_edge, b_edge, Ws0, Wd0):
    n, _ = atom_fea.shape
    e = edge_attr.shape[0]
    bn = 200
    be = bn * 32
    grid = n // bn

    def body(af, ea, wa, ba, we, beb, ws, wd, node_o, ps_o, pd_o, edge_o, mf_o):
        nd = jnp.dot(af[...], wa[...], precision=_PREC) + ba[...]
        node_o[...] = nd
        ps_o[...] = jnp.dot(nd, ws[...], precision=_PREC)
        pd_o[...] = jnp.dot(nd, wd[...], precision=_PREC)
        ea_v = ea[...]
        edge_o[...] = jnp.dot(ea_v, we[...], precision=_PREC) + beb[...]
        mf_o[...] = (jnp.sum(jnp.abs(ea_v), axis=1, keepdims=True)
                     > 1e-06).astype(_F32)

    return pl.pallas_call(
        body,
        grid=(grid,),
        in_specs=[
            pl.BlockSpec((bn, 128), lambda i: (i, 0)),
            pl.BlockSpec((be, 16), lambda i: (i, 0)),
            pl.BlockSpec((128, 64), lambda i: (0, 0)),
            pl.BlockSpec((1, 64), lambda i: (0, 0)),
            pl.BlockSpec((16, 16), lambda i: (0, 0)),
            pl.BlockSpec((1, 16), lambda i: (0, 0)),
            pl.BlockSpec((64, 64), lambda i: (0, 0)),
            pl.BlockSpec((64, 64), lambda i: (0, 0)),
        ],
        out_specs=[
            pl.BlockSpec((bn, 64), lambda i: (i, 0)),
            pl.BlockSpec((bn, 64), lambda i: (i, 0)),
            pl.BlockSpec((bn, 64), lambda i: (i, 0)),
            pl.BlockSpec((be, 16), lambda i: (i, 0)),
            pl.BlockSpec((be, 1), lambda i: (i, 0)),
        ],
        out_shape=[
            jax.ShapeDtypeStruct((n, 64), _F32),
            jax.ShapeDtypeStruct((n, 64), _F32),
            jax.ShapeDtypeStruct((n, 64), _F32),
            jax.ShapeDtypeStruct((e, 16), _F32),
            jax.ShapeDtypeStruct((e, 1), _F32),
        ],
    )(atom_fea, edge_attr, W_atom, b_atom, W_edge, b_edge, Ws0, Wd0)


def _tc_edge(edge, G, ps, maskf, U, b1, W2, b2):
    e = edge.shape[0]
    n = ps.shape[0]
    bn = 200
    be = bn * 32
    grid = n // bn

    def body(e_ref, g_ref, ps_ref, mf_ref, u_ref, b1_ref, w2_ref, b2_ref,
             eo_ref, mo_ref):
        psb = jnp.broadcast_to(ps_ref[...][:, None, :], (bn, 32, 64))
        psr = psb.reshape(be, 64)
        ev = e_ref[...]
        pre = (jnp.dot(ev, u_ref[...], precision=_PREC) + psr + g_ref[...]
               + b1_ref[...])
        h = _silu(pre)
        enew = ev + jnp.dot(h, w2_ref[...], precision=_PREC) + b2_ref[...]
        eo_ref[...] = enew
        mo_ref[...] = enew * mf_ref[...]

    return pl.pallas_call(
        body,
        grid=(grid,),
        in_specs=[
            pl.BlockSpec((be, 16), lambda i: (i, 0)),
            pl.BlockSpec((be, 64), lambda i: (i, 0)),
            pl.BlockSpec((bn, 64), lambda i: (i, 0)),
            pl.BlockSpec((be, 1), lambda i: (i, 0)),
            pl.BlockSpec((16, 64), lambda i: (0, 0)),
            pl.BlockSpec((1, 64), lambda i: (0, 0)),
            pl.BlockSpec((64, 16), lambda i: (0, 0)),
            pl.BlockSpec((1, 16), lambda i: (0, 0)),
        ],
        out_specs=[
            pl.BlockSpec((be, 16), lambda i: (i, 0)),
            pl.BlockSpec((be, 16), lambda i: (i, 0)),
        ],
        out_shape=[
            jax.ShapeDtypeStruct((e, 16), _F32),
            jax.ShapeDtypeStruct((e, 16), _F32),
        ],
    )(edge, G, ps, maskf, U, b1, W2, b2)


def _tc_node(node, aggP, rinv, Wn1a, Wn1b, bn1, Wn2, bn2, Ws, Wd):
    n = node.shape[0]
    bn = 2000
    grid = n // bn

    def body(nd_ref, ag_ref, ri_ref, w1a, w1b, b1r, w2r, b2r, wsr, wdr,
             no_ref, ps_ref, pd_ref):
        agv = ag_ref[...]
        agg = (agv[0] + agv[1]) * ri_ref[...]
        nd = nd_ref[...]
        h = _silu(jnp.dot(nd, w1a[...], precision=_PREC)
                  + jnp.dot(agg, w1b[...], precision=_PREC) + b1r[...])
        nn = nd + jnp.dot(h, w2r[...], precision=_PREC) + b2r[...]
        no_ref[...] = nn
        ps_ref[...] = jnp.dot(nn, wsr[...], precision=_PREC)
        pd_ref[...] = jnp.dot(nn, wdr[...], precision=_PREC)

    return pl.pallas_call(
        body,
        grid=(grid,),
        in_specs=[
            pl.BlockSpec((bn, 64), lambda i: (i, 0)),
            pl.BlockSpec((2, bn, 16), lambda i: (0, i, 0)),
            pl.BlockSpec((bn, 1), lambda i: (i, 0)),
            pl.BlockSpec((64, 64), lambda i: (0, 0)),
            pl.BlockSpec((16, 64), lambda i: (0, 0)),
            pl.BlockSpec((1, 64), lambda i: (0, 0)),
            pl.BlockSpec((64, 64), lambda i: (0, 0)),
            pl.BlockSpec((1, 64), lambda i: (0, 0)),
            pl.BlockSpec((64, 64), lambda i: (0, 0)),
            pl.BlockSpec((64, 64), lambda i: (0, 0)),
        ],
        out_specs=[
            pl.BlockSpec((bn, 64), lambda i: (i, 0)),
            pl.BlockSpec((bn, 64), lambda i: (i, 0)),
            pl.BlockSpec((bn, 64), lambda i: (i, 0)),
        ],
        out_shape=[
            jax.ShapeDtypeStruct((n, 64), _F32),
            jax.ShapeDtypeStruct((n, 64), _F32),
            jax.ShapeDtypeStruct((n, 64), _F32),
        ],
    )(node, aggP, rinv, Wn1a, Wn1b, bn1, Wn2, bn2, Ws, Wd)


def _tc_rinv(cntP):
    nw, n = cntP.shape

    def body(c_ref, o_ref):
        cnt = jnp.sum(c_ref[...], axis=0)
        o_ref[...] = 1.0 / jnp.maximum(cnt, 1.0)

    return pl.pallas_call(
        body,
        grid=(1,),
        in_specs=[pl.BlockSpec((nw, n), lambda i: (0, 0))],
        out_specs=pl.BlockSpec((n,), lambda i: (0,)),
        out_shape=jax.ShapeDtypeStruct((n,), _F32),
    )(cntP)


def _tc_readout(R, A, W_r, b_r, W_o, b_o):
    b = A.shape[0]
    ep = R.shape[0]

    def body(r_ref, a_ref, wr, br, wo, bo, o_ref):
        crys = jnp.dot(a_ref[...], r_ref[...], precision=_PREC)
        cr = _silu(jnp.dot(crys, wr[...], precision=_PREC) + br[...])
        o_ref[...] = jnp.dot(cr, wo[...], precision=_PREC) + bo[...]

    return pl.pallas_call(
        body,
        grid=(1,),
        in_specs=[
            pl.BlockSpec((ep, 64), lambda i: (0, 0)),
            pl.BlockSpec((b, ep), lambda i: (0, 0)),
            pl.BlockSpec((64, 128), lambda i: (0, 0)),
            pl.BlockSpec((1, 128), lambda i: (0, 0)),
            pl.BlockSpec((128, 1), lambda i: (0, 0)),
            pl.BlockSpec((1, 1), lambda i: (0, 0)),
        ],
        out_specs=pl.BlockSpec((b, 1), lambda i: (0, 0)),
        out_shape=jax.ShapeDtypeStruct((b, 1), _F32),
    )(R, A, W_r, b_r, W_o, b_o)


def kernel(atom_fea, nbr_fea, nbr_fea_idx, crystal_atom_idx, W_atom, b_atom,
           W_edge, b_edge, W_e1, b_e1, W_e2, b_e2, W_n1, b_n1, W_n2, b_n2,
           W_r, b_r, W_o, b_o):
    n, m = nbr_fea_idx.shape
    e = n * m
    nl = W_e1.shape[0]
    b, p = crystal_atom_idx.shape

    edge_attr = nbr_fea.reshape(e, nbr_fea.shape[-1])
    dst = jnp.clip(nbr_fea_idx.reshape(e), 0, n - 1).astype(jnp.int32)
    zinit = jnp.zeros((n, 16), _F32)
    zeros1d = jnp.zeros((n,), _F32)

    node, ps, pd, edge, maskf = _tc_init(
        atom_fea, edge_attr, W_atom, b_atom.reshape(1, 64), W_edge,
        b_edge.reshape(1, 16), W_e1[0, 16:80], W_e1[0, 80:144])

    cntP = _sc_counts(maskf.reshape(e), dst, zeros1d, 2000)
    rinv = _tc_rinv(cntP).reshape(n, 1)

    for l in range(nl):
        G = _sc_gather(pd, dst, 1000)
        edge, masked = _tc_edge(edge, G, ps, maskf, W_e1[l, :16],
                                b_e1[l].reshape(1, 64), W_e2[l],
                                b_e2[l].reshape(1, 16))
        aggP = _sc_scatter_add(masked, dst, zinit, 2000)
        ln = (l + 1) % nl
        node, ps, pd = _tc_node(node, aggP, rinv, W_n1[l, :64], W_n1[l, 64:80],
                                b_n1[l].reshape(1, 64), W_n2[l],
                                b_n2[l].reshape(1, 64), W_e1[ln, 16:80],
                                W_e1[ln, 80:144])

    ep = ((b * p + 255) // 256) * 256
    cai = jnp.concatenate([crystal_atom_idx.reshape(b * p).astype(jnp.int32),
                           jnp.zeros((ep - b * p,), jnp.int32)])
    R = _sc_gather(node, cai, ep // 32)
    col = jnp.arange(ep)
    avg = ((col[None, :] // p == jnp.arange(b)[:, None])
           & (col[None, :] < b * p)).astype(_F32) / p
    return _tc_readout(R, avg, W_r, b_r.reshape(1, 128), W_o,
                       b_o.reshape(1, 1))


# speedup vs baseline: 3.5242x; 1.4312x over previous
import functools

import jax
import jax.numpy as jnp
from jax import lax
from jax.experimental import pallas as pl
from jax.experimental.pallas import tpu as pltpu
from jax.experimental.pallas import tpu_sc as plsc

_F32 = jnp.float32
_PREC = jax.lax.Precision.HIGHEST


def _silu(x):
    return x * jax.nn.sigmoid(x)


def _mm(a, b):
    return jnp.dot(a.astype(jnp.bfloat16), b.astype(jnp.bfloat16),
                   preferred_element_type=jnp.float32)


def _sc_gather(table, idx, chunk):
    n_rows, d = table.shape
    e = idx.shape[0]
    nw = 32
    epw = e // nw
    nch = epw // chunk
    mesh = plsc.VectorSubcoreMesh(core_axis_name="c", subcore_axis_name="s")

    @functools.partial(
        pl.kernel,
        mesh=mesh,
        compiler_params=pltpu.CompilerParams(use_tc_tiling_on_sc=False, needs_layout_passes=False),
        out_type=jax.ShapeDtypeStruct((e, d), _F32),
        scratch_types=[
            pltpu.VMEM((chunk,), jnp.int32),
            pltpu.VMEM((chunk, d), _F32),
            pltpu.SemaphoreType.DMA,
        ],
    )
    def k(table_hbm, idx_hbm, out_hbm, idx_v, rows_v, sem):
        wid = lax.axis_index("s") * 2 + lax.axis_index("c")

        def body(c, carry):
            base = wid * epw + c * chunk
            pltpu.sync_copy(idx_hbm.at[pl.ds(base, chunk)], idx_v)
            pltpu.async_copy(table_hbm.at[idx_v], rows_v, sem).wait()
            pltpu.sync_copy(rows_v, out_hbm.at[pl.ds(base, chunk)])
            return carry

        lax.fori_loop(0, nch, body, 0)

    return k(table, idx)


def _sc_scatter_add(vals, idx, zinit, chunk):
    e = vals.shape[0]
    n_rows = zinit.shape[0]
    nw = 32
    epw = e // nw
    nch = epw // chunk
    rps = n_rows // 16
    mesh = plsc.VectorSubcoreMesh(core_axis_name="c", subcore_axis_name="s")

    @functools.partial(
        pl.kernel,
        mesh=mesh,
        compiler_params=pltpu.CompilerParams(use_tc_tiling_on_sc=False, needs_layout_passes=False),
        out_type=jax.ShapeDtypeStruct((2, n_rows, 16), _F32),
        scratch_types=[
            pltpu.VMEM((chunk,), jnp.int32),
            pltpu.VMEM((chunk, 16), _F32),
            pltpu.VMEM_SHARED((n_rows, 16), _F32),
            pltpu.SemaphoreType.DMA,
        ],
    )
    def k(vals_hbm, idx_hbm, zinit_hbm, out_hbm, idx_v, vals_v, shared, sem):
        cid = lax.axis_index("c")
        sid = lax.axis_index("s")
        wid = sid * 2 + cid
        pltpu.sync_copy(zinit_hbm.at[pl.ds(sid * rps, rps)],
                        vals_v.at[pl.ds(0, rps)])
        pltpu.sync_copy(vals_v.at[pl.ds(0, rps)],
                        shared.at[pl.ds(sid * rps, rps)])
        plsc.subcore_barrier()

        def body(c, carry):
            base = wid * epw + c * chunk
            pltpu.sync_copy(idx_hbm.at[pl.ds(base, chunk)], idx_v)
            pltpu.sync_copy(vals_hbm.at[pl.ds(base, chunk)], vals_v)
            pltpu.sync_copy(vals_v, shared.at[idx_v], add=True)
            return carry

        lax.fori_loop(0, nch, body, 0)
        plsc.subcore_barrier()
        pltpu.sync_copy(shared.at[pl.ds(sid * rps, rps)],
                        vals_v.at[pl.ds(0, rps)])
        pltpu.sync_copy(vals_v.at[pl.ds(0, rps)],
                        out_hbm.at[cid, pl.ds(sid * rps, rps)])

    return k(vals, idx, zinit)


def _sc_counts(maskf, idx, zeros1d, chunk):
    e = maskf.shape[0]
    n_rows = zeros1d.shape[0]
    nw = 32
    epw = e // nw
    nch = epw // chunk
    mesh = plsc.VectorSubcoreMesh(core_axis_name="c", subcore_axis_name="s")

    @functools.partial(
        pl.kernel,
        mesh=mesh,
        compiler_params=pltpu.CompilerParams(use_tc_tiling_on_sc=False, needs_layout_passes=False),
        out_type=jax.ShapeDtypeStruct((nw, n_rows), _F32),
        scratch_types=[
            pltpu.VMEM((chunk,), jnp.int32),
            pltpu.VMEM((chunk,), _F32),
            pltpu.VMEM((n_rows,), _F32),
        ],
    )
    def k(mask_hbm, idx_hbm, z_hbm, out_hbm, idx_v, mask_v, cnt_v):
        wid = lax.axis_index("s") * 2 + lax.axis_index("c")
        pltpu.sync_copy(z_hbm, cnt_v)

        def chunk_body(c, carry):
            base = wid * epw + c * chunk
            pltpu.sync_copy(idx_hbm.at[pl.ds(base, chunk)], idx_v)
            pltpu.sync_copy(mask_hbm.at[pl.ds(base, chunk)], mask_v)

            def vec_body(j, carry2):
                i16 = idx_v[pl.ds(j * 16, 16)]
                m16 = mask_v[pl.ds(j * 16, 16)]
                plsc.addupdate_scatter(cnt_v, [i16], m16)
                return carry2

            lax.fori_loop(0, chunk // 16, vec_body, 0)
            return carry

        lax.fori_loop(0, nch, chunk_body, 0)
        pltpu.sync_copy(cnt_v, out_hbm.at[wid])

    return k(maskf, idx, zeros1d)


def _tc_init(atom_fea, edge_attr, W_atom, b_atom, W_edge, b_edge, Ws0, Wd0):
    n, _ = atom_fea.shape
    e = edge_attr.shape[0]
    bn = 200
    be = bn * 32
    grid = n // bn

    def body(af, ea, wa, ba, we, beb, ws, wd, node_o, ps_o, pd_o, edge_o, mf_o):
        nd = _mm(af[...], wa[...]) + ba[...]
        node_o[...] = nd
        ps_o[...] = _mm(nd, ws[...])
        pd_o[...] = _mm(nd, wd[...])
        ea_v = ea[...]
        edge_o[...] = _mm(ea_v, we[...]) + beb[...]
        mf_o[...] = (jnp.sum(jnp.abs(ea_v), axis=1, keepdims=True)
                     > 1e-06).astype(_F32)

    return pl.pallas_call(
        body,
        grid=(grid,),
        in_specs=[
            pl.BlockSpec((bn, 128), lambda i: (i, 0)),
            pl.BlockSpec((be, 16), lambda i: (i, 0)),
            pl.BlockSpec((128, 64), lambda i: (0, 0)),
            pl.BlockSpec((1, 64), lambda i: (0, 0)),
            pl.BlockSpec((16, 16), lambda i: (0, 0)),
            pl.BlockSpec((1, 16), lambda i: (0, 0)),
            pl.BlockSpec((64, 64), lambda i: (0, 0)),
            pl.BlockSpec((64, 64), lambda i: (0, 0)),
        ],
        out_specs=[
            pl.BlockSpec((bn, 64), lambda i: (i, 0)),
            pl.BlockSpec((bn, 64), lambda i: (i, 0)),
            pl.BlockSpec((bn, 64), lambda i: (i, 0)),
            pl.BlockSpec((be, 16), lambda i: (i, 0)),
            pl.BlockSpec((be, 1), lambda i: (i, 0)),
        ],
        out_shape=[
            jax.ShapeDtypeStruct((n, 64), _F32),
            jax.ShapeDtypeStruct((n, 64), _F32),
            jax.ShapeDtypeStruct((n, 64), _F32),
            jax.ShapeDtypeStruct((e, 16), _F32),
            jax.ShapeDtypeStruct((e, 1), _F32),
        ],
    )(atom_fea, edge_attr, W_atom, b_atom, W_edge, b_edge, Ws0, Wd0)


def _tc_edge(edge, G, ps, maskf, U, b1, W2, b2):
    e = edge.shape[0]
    n = ps.shape[0]
    bn = 200
    be = bn * 32
    grid = n // bn

    def body(e_ref, g_ref, ps_ref, mf_ref, u_ref, b1_ref, w2_ref, b2_ref,
             eo_ref, mo_ref):
        psb = jnp.broadcast_to(ps_ref[...][:, None, :], (bn, 32, 64))
        psr = psb.reshape(be, 64)
        ev = e_ref[...]
        pre = (_mm(ev, u_ref[...]) + psr + g_ref[...]
               + b1_ref[...])
        h = _silu(pre)
        enew = ev + _mm(h, w2_ref[...]) + b2_ref[...]
        eo_ref[...] = enew
        mo_ref[...] = enew * mf_ref[...]

    return pl.pallas_call(
        body,
        grid=(grid,),
        in_specs=[
            pl.BlockSpec((be, 16), lambda i: (i, 0)),
            pl.BlockSpec((be, 64), lambda i: (i, 0)),
            pl.BlockSpec((bn, 64), lambda i: (i, 0)),
            pl.BlockSpec((be, 1), lambda i: (i, 0)),
            pl.BlockSpec((16, 64), lambda i: (0, 0)),
            pl.BlockSpec((1, 64), lambda i: (0, 0)),
            pl.BlockSpec((64, 16), lambda i: (0, 0)),
            pl.BlockSpec((1, 16), lambda i: (0, 0)),
        ],
        out_specs=[
            pl.BlockSpec((be, 16), lambda i: (i, 0)),
            pl.BlockSpec((be, 16), lambda i: (i, 0)),
        ],
        out_shape=[
            jax.ShapeDtypeStruct((e, 16), _F32),
            jax.ShapeDtypeStruct((e, 16), _F32),
        ],
    )(edge, G, ps, maskf, U, b1, W2, b2)


def _tc_node(node, aggP, rinv, Wn1a, Wn1b, bn1, Wn2, bn2, Ws, Wd):
    n = node.shape[0]
    bn = 2000
    grid = n // bn

    def body(nd_ref, ag_ref, ri_ref, w1a, w1b, b1r, w2r, b2r, wsr, wdr,
             no_ref, ps_ref, pd_ref):
        agv = ag_ref[...]
        agg = (agv[0] + agv[1]) * ri_ref[...]
        nd = nd_ref[...]
        h = _silu(_mm(nd, w1a[...])
                  + _mm(agg, w1b[...]) + b1r[...])
        nn = nd + _mm(h, w2r[...]) + b2r[...]
        no_ref[...] = nn
        ps_ref[...] = _mm(nn, wsr[...])
        pd_ref[...] = _mm(nn, wdr[...])

    return pl.pallas_call(
        body,
        grid=(grid,),
        in_specs=[
            pl.BlockSpec((bn, 64), lambda i: (i, 0)),
            pl.BlockSpec((2, bn, 16), lambda i: (0, i, 0)),
            pl.BlockSpec((bn, 1), lambda i: (i, 0)),
            pl.BlockSpec((64, 64), lambda i: (0, 0)),
            pl.BlockSpec((16, 64), lambda i: (0, 0)),
            pl.BlockSpec((1, 64), lambda i: (0, 0)),
            pl.BlockSpec((64, 64), lambda i: (0, 0)),
            pl.BlockSpec((1, 64), lambda i: (0, 0)),
            pl.BlockSpec((64, 64), lambda i: (0, 0)),
            pl.BlockSpec((64, 64), lambda i: (0, 0)),
        ],
        out_specs=[
            pl.BlockSpec((bn, 64), lambda i: (i, 0)),
            pl.BlockSpec((bn, 64), lambda i: (i, 0)),
            pl.BlockSpec((bn, 64), lambda i: (i, 0)),
        ],
        out_shape=[
            jax.ShapeDtypeStruct((n, 64), _F32),
            jax.ShapeDtypeStruct((n, 64), _F32),
            jax.ShapeDtypeStruct((n, 64), _F32),
        ],
    )(node, aggP, rinv, Wn1a, Wn1b, bn1, Wn2, bn2, Ws, Wd)


def _tc_rinv(cntP):
    nw, n = cntP.shape

    def body(c_ref, o_ref):
        cnt = jnp.sum(c_ref[...], axis=0)
        o_ref[...] = 1.0 / jnp.maximum(cnt, 1.0)

    return pl.pallas_call(
        body,
        grid=(1,),
        in_specs=[pl.BlockSpec((nw, n), lambda i: (0, 0))],
        out_specs=pl.BlockSpec((n,), lambda i: (0,)),
        out_shape=jax.ShapeDtypeStruct((n,), _F32),
    )(cntP)


def _tc_readout(R, A, W_r, b_r, W_o, b_o):
    b = A.shape[0]
    ep = R.shape[0]

    def body(r_ref, a_ref, wr, br, wo, bo, o_ref):
        crys = _mm(a_ref[...], r_ref[...])
        cr = _silu(_mm(crys, wr[...]) + br[...])
        o_ref[...] = _mm(cr, wo[...]) + bo[...]

    return pl.pallas_call(
        body,
        grid=(1,),
        in_specs=[
            pl.BlockSpec((ep, 64), lambda i: (0, 0)),
            pl.BlockSpec((b, ep), lambda i: (0, 0)),
            pl.BlockSpec((64, 128), lambda i: (0, 0)),
            pl.BlockSpec((1, 128), lambda i: (0, 0)),
            pl.BlockSpec((128, 1), lambda i: (0, 0)),
            pl.BlockSpec((1, 1), lambda i: (0, 0)),
        ],
        out_specs=pl.BlockSpec((b, 1), lambda i: (0, 0)),
        out_shape=jax.ShapeDtypeStruct((b, 1), _F32),
    )(R, A, W_r, b_r, W_o, b_o)


def kernel(atom_fea, nbr_fea, nbr_fea_idx, crystal_atom_idx, W_atom, b_atom,
           W_edge, b_edge, W_e1, b_e1, W_e2, b_e2, W_n1, b_n1, W_n2, b_n2,
           W_r, b_r, W_o, b_o):
    n, m = nbr_fea_idx.shape
    e = n * m
    nl = W_e1.shape[0]
    b, p = crystal_atom_idx.shape

    edge_attr = nbr_fea.reshape(e, nbr_fea.shape[-1])
    dst = jnp.clip(nbr_fea_idx.reshape(e), 0, n - 1).astype(jnp.int32)
    zinit = jnp.zeros((n, 16), _F32)
    zeros1d = jnp.zeros((n,), _F32)

    node, ps, pd, edge, maskf = _tc_init(
        atom_fea, edge_attr, W_atom, b_atom.reshape(1, 64), W_edge,
        b_edge.reshape(1, 16), W_e1[0, 16:80], W_e1[0, 80:144])

    cntP = _sc_counts(maskf.reshape(e), dst, zeros1d, 2000)
    rinv = _tc_rinv(cntP).reshape(n, 1)

    for l in range(nl):
        G = _sc_gather(pd, dst, 1000)
        edge, masked = _tc_edge(edge, G, ps, maskf, W_e1[l, :16],
                                b_e1[l].reshape(1, 64), W_e2[l],
                                b_e2[l].reshape(1, 16))
        aggP = _sc_scatter_add(masked, dst, zinit, 2000)
        ln = (l + 1) % nl
        node, ps, pd = _tc_node(node, aggP, rinv, W_n1[l, :64], W_n1[l, 64:80],
                                b_n1[l].reshape(1, 64), W_n2[l],
                                b_n2[l].reshape(1, 64), W_e1[ln, 16:80],
                                W_e1[ln, 80:144])

    ep = ((b * p + 255) // 256) * 256
    cai = jnp.concatenate([crystal_atom_idx.reshape(b * p).astype(jnp.int32),
                           jnp.zeros((ep - b * p,), jnp.int32)])
    R = _sc_gather(node, cai, ep // 32)
    col = jnp.arange(ep)
    avg = ((col[None, :] // p == jnp.arange(b)[:, None])
           & (col[None, :] < b * p)).astype(_F32) / p
    return _tc_readout(R, avg, W_r, b_r.reshape(1, 128), W_o,
                       b_o.reshape(1, 1))


# speedup vs baseline: 9.3167x; 2.6436x over previous
import functools

import jax
import jax.numpy as jnp
from jax import lax
from jax.experimental import pallas as pl
from jax.experimental.pallas import tpu as pltpu
from jax.experimental.pallas import tpu_sc as plsc

_F32 = jnp.float32
_BF16 = jnp.bfloat16


def _silu(x):
    return x * jax.nn.sigmoid(x)


def _mm(a, b):
    return jnp.dot(a.astype(_BF16), b.astype(_BF16),
                   preferred_element_type=_F32)


def _kron8(w):
    return jnp.kron(jnp.eye(8, dtype=_F32), w)


def _sc_gather(table, idx, chunk):
    n_rows, d = table.shape
    e = idx.shape[0]
    nw = 32
    epw = e // nw
    nch = epw // chunk
    mesh = plsc.VectorSubcoreMesh(core_axis_name="c", subcore_axis_name="s")

    @functools.partial(
        pl.kernel,
        mesh=mesh,
        compiler_params=pltpu.CompilerParams(use_tc_tiling_on_sc=False,
                                             needs_layout_passes=False),
        out_type=jax.ShapeDtypeStruct((e, d), table.dtype),
        scratch_types=[
            pltpu.VMEM((chunk,), jnp.int32),
            pltpu.VMEM((chunk, d), table.dtype),
            pltpu.SemaphoreType.DMA,
        ],
    )
    def k(table_hbm, idx_hbm, out_hbm, idx_v, rows_v, sem):
        wid = lax.axis_index("s") * 2 + lax.axis_index("c")

        def body(c, carry):
            base = wid * epw + c * chunk
            pltpu.sync_copy(idx_hbm.at[pl.ds(base, chunk)], idx_v)
            pltpu.async_copy(table_hbm.at[idx_v], rows_v, sem).wait()
            pltpu.sync_copy(rows_v, out_hbm.at[pl.ds(base, chunk)])
            return carry

        lax.fori_loop(0, nch, body, 0)

    return k(table, idx)


def _sc_scatter_add(vals, idx, zinit, chunk):
    e = vals.shape[0]
    n_rows = zinit.shape[0]
    nw = 32
    epw = e // nw
    nch = epw // chunk
    rps = n_rows // 16
    mesh = plsc.VectorSubcoreMesh(core_axis_name="c", subcore_axis_name="s")

    @functools.partial(
        pl.kernel,
        mesh=mesh,
        compiler_params=pltpu.CompilerParams(use_tc_tiling_on_sc=False,
                                             needs_layout_passes=False),
        out_type=jax.ShapeDtypeStruct((2, n_rows, 16), _F32),
        scratch_types=[
            pltpu.VMEM((chunk,), jnp.int32),
            pltpu.VMEM((chunk, 16), _F32),
            pltpu.VMEM_SHARED((n_rows, 16), _F32),
            pltpu.SemaphoreType.DMA,
        ],
    )
    def k(vals_hbm, idx_hbm, zinit_hbm, out_hbm, idx_v, vals_v, shared, sem):
        cid = lax.axis_index("c")
        sid = lax.axis_index("s")
        wid = sid * 2 + cid
        pltpu.sync_copy(zinit_hbm.at[pl.ds(sid * rps, rps)],
                        vals_v.at[pl.ds(0, rps)])
        pltpu.sync_copy(vals_v.at[pl.ds(0, rps)],
                        shared.at[pl.ds(sid * rps, rps)])
        plsc.subcore_barrier()

        def body(c, carry):
            base = wid * epw + c * chunk
            pltpu.sync_copy(idx_hbm.at[pl.ds(base, chunk)], idx_v)
            pltpu.sync_copy(vals_hbm.at[pl.ds(base, chunk)], vals_v)
            pltpu.sync_copy(vals_v, shared.at[idx_v], add=True)
            return carry

        lax.fori_loop(0, nch, body, 0)
        plsc.subcore_barrier()
        pltpu.sync_copy(shared.at[pl.ds(sid * rps, rps)],
                        vals_v.at[pl.ds(0, rps)])
        pltpu.sync_copy(vals_v.at[pl.ds(0, rps)],
                        out_hbm.at[cid, pl.ds(sid * rps, rps)])

    return k(vals, idx, zinit)


def _tc_init(atom_fea, edge_attr_p, W_atom, b_atom, K8We, b_edge8, K8ones,
             Ws0, Wd0):
    n, _ = atom_fea.shape
    e8 = edge_attr_p.shape[0]
    bn = 400
    b8 = bn * 4
    grid = n // bn

    def body(af, ea, wa, ba, we, beb, ko, ws, wd, node_o, ps_o, pd_o, edge_o,
             mf_o):
        nd = _mm(af[...], wa[...]) + ba[...]
        node_o[...] = nd
        ps_o[...] = _mm(nd, ws[...])
        pd_o[...] = _mm(nd, wd[...])
        ea_v = ea[...]
        edge_o[...] = _mm(ea_v, we[...]) + beb[...]
        gsum = _mm(jnp.abs(ea_v), ko[...])
        mf_o[...] = (gsum > 1e-06).astype(_F32)

    return pl.pallas_call(
        body,
        grid=(grid,),
        in_specs=[
            pl.BlockSpec((bn, 128), lambda i: (i, 0)),
            pl.BlockSpec((b8, 128), lambda i: (i, 0)),
            pl.BlockSpec((128, 64), lambda i: (0, 0)),
            pl.BlockSpec((1, 64), lambda i: (0, 0)),
            pl.BlockSpec((128, 128), lambda i: (0, 0)),
            pl.BlockSpec((1, 128), lambda i: (0, 0)),
            pl.BlockSpec((128, 128), lambda i: (0, 0)),
            pl.BlockSpec((64, 64), lambda i: (0, 0)),
            pl.BlockSpec((64, 64), lambda i: (0, 0)),
        ],
        out_specs=[
            pl.BlockSpec((bn, 64), lambda i: (i, 0)),
            pl.BlockSpec((bn, 64), lambda i: (i, 0)),
            pl.BlockSpec((bn, 64), lambda i: (i, 0)),
            pl.BlockSpec((b8, 128), lambda i: (i, 0)),
            pl.BlockSpec((b8, 128), lambda i: (i, 0)),
        ],
        out_shape=[
            jax.ShapeDtypeStruct((n, 64), _F32),
            jax.ShapeDtypeStruct((n, 64), _F32),
            jax.ShapeDtypeStruct((n, 64), _F32),
            jax.ShapeDtypeStruct((e8, 128), _F32),
            jax.ShapeDtypeStruct((e8, 128), _F32),
        ],
    )(atom_fea, edge_attr_p, W_atom, b_atom, K8We, b_edge8, K8ones, Ws0, Wd0)


def _tc_edge(edge_p, G2, ps, mask_p, K8U, b18, K8W2, b28):
    e8 = edge_p.shape[0]
    n = ps.shape[0]
    bn = 400
    b8 = bn * 4
    b2 = bn * 16
    grid = n // bn

    def body(e_ref, g_ref, ps_ref, mf_ref, u_ref, b1_ref, w2_ref, b2_ref,
             eo_ref, mo_ref):
        psl = jnp.tile(ps_ref[...], (1, 8))
        psb = jnp.broadcast_to(psl[:, None, :], (bn, 4, 512))
        psr = psb.reshape(b8, 512)
        g8 = g_ref[...].reshape(b8, 512)
        ev = e_ref[...]
        pre = _mm(ev, u_ref[...]) + psr + g8 + b1_ref[...]
        h = _silu(pre)
        enew = ev + _mm(h, w2_ref[...]) + b2_ref[...]
        eo_ref[...] = enew
        mo_ref[...] = enew * mf_ref[...]

    return pl.pallas_call(
        body,
        grid=(grid,),
        in_specs=[
            pl.BlockSpec((b8, 128), lambda i: (i, 0)),
            pl.BlockSpec((b2, 128), lambda i: (i, 0)),
            pl.BlockSpec((bn, 64), lambda i: (i, 0)),
            pl.BlockSpec((b8, 128), lambda i: (i, 0)),
            pl.BlockSpec((128, 512), lambda i: (0, 0)),
            pl.BlockSpec((1, 512), lambda i: (0, 0)),
            pl.BlockSpec((512, 128), lambda i: (0, 0)),
            pl.BlockSpec((1, 128), lambda i: (0, 0)),
        ],
        out_specs=[
            pl.BlockSpec((b8, 128), lambda i: (i, 0)),
            pl.BlockSpec((b8, 128), lambda i: (i, 0)),
        ],
        out_shape=[
            jax.ShapeDtypeStruct((e8, 128), _F32),
            jax.ShapeDtypeStruct((e8, 128), _F32),
        ],
    )(edge_p, G2, ps, mask_p, K8U, b18, K8W2, b28)


def _tc_node(node, aggP, rinv, Wn1a, Wn1b, bn1, Wn2, bn2, Ws, Wd):
    n = node.shape[0]
    bn = 2000
    grid = n // bn

    def body(nd_ref, ag_ref, ri_ref, w1a, w1b, b1r, w2r, b2r, wsr, wdr,
             no_ref, ps_ref, pd_ref):
        agv = ag_ref[...]
        agg = (agv[0] + agv[1]) * ri_ref[...]
        nd = nd_ref[...]
        h = _silu(_mm(nd, w1a[...]) + _mm(agg, w1b[...]) + b1r[...])
        nn = nd + _mm(h, w2r[...]) + b2r[...]
        no_ref[...] = nn
        ps_ref[...] = _mm(nn, wsr[...])
        pd_ref[...] = _mm(nn, wdr[...])

    return pl.pallas_call(
        body,
        grid=(grid,),
        in_specs=[
            pl.BlockSpec((bn, 64), lambda i: (i, 0)),
            pl.BlockSpec((2, bn, 16), lambda i: (0, i, 0)),
            pl.BlockSpec((bn, 16), lambda i: (i, 0)),
            pl.BlockSpec((64, 64), lambda i: (0, 0)),
            pl.BlockSpec((16, 64), lambda i: (0, 0)),
            pl.BlockSpec((1, 64), lambda i: (0, 0)),
            pl.BlockSpec((64, 64), lambda i: (0, 0)),
            pl.BlockSpec((1, 64), lambda i: (0, 0)),
            pl.BlockSpec((64, 64), lambda i: (0, 0)),
            pl.BlockSpec((64, 64), lambda i: (0, 0)),
        ],
        out_specs=[
            pl.BlockSpec((bn, 64), lambda i: (i, 0)),
            pl.BlockSpec((bn, 64), lambda i: (i, 0)),
            pl.BlockSpec((bn, 64), lambda i: (i, 0)),
        ],
        out_shape=[
            jax.ShapeDtypeStruct((n, 64), _F32),
            jax.ShapeDtypeStruct((n, 64), _F32),
            jax.ShapeDtypeStruct((n, 64), _F32),
        ],
    )(node, aggP, rinv, Wn1a, Wn1b, bn1, Wn2, bn2, Ws, Wd)


def _tc_rinv(cntP):
    _, n, _ = cntP.shape
    bn = 2000
    grid = n // bn

    def body(c_ref, o_ref):
        cv = c_ref[...]
        cnt = cv[0] + cv[1]
        o_ref[...] = 1.0 / jnp.maximum(cnt, 1.0)

    return pl.pallas_call(
        body,
        grid=(grid,),
        in_specs=[pl.BlockSpec((2, bn, 16), lambda i: (0, i, 0))],
        out_specs=pl.BlockSpec((bn, 16), lambda i: (i, 0)),
        out_shape=jax.ShapeDtypeStruct((n, 16), _F32),
    )(cntP)


def _tc_readout(R, A, W_r, b_r, W_o, b_o):
    b = A.shape[0]
    ep = R.shape[0]

    def body(r_ref, a_ref, wr, br, wo, bo, o_ref):
        crys = _mm(a_ref[...], r_ref[...])
        cr = _silu(_mm(crys, wr[...]) + br[...])
        o_ref[...] = _mm(cr, wo[...]) + bo[...]

    return pl.pallas_call(
        body,
        grid=(1,),
        in_specs=[
            pl.BlockSpec((ep, 64), lambda i: (0, 0)),
            pl.BlockSpec((b, ep), lambda i: (0, 0)),
            pl.BlockSpec((64, 128), lambda i: (0, 0)),
            pl.BlockSpec((1, 128), lambda i: (0, 0)),
            pl.BlockSpec((128, 1), lambda i: (0, 0)),
            pl.BlockSpec((1, 1), lambda i: (0, 0)),
        ],
        out_specs=pl.BlockSpec((b, 1), lambda i: (0, 0)),
        out_shape=jax.ShapeDtypeStruct((b, 1), _F32),
    )(R, A, W_r, b_r, W_o, b_o)


def kernel(atom_fea, nbr_fea, nbr_fea_idx, crystal_atom_idx, W_atom, b_atom,
           W_edge, b_edge, W_e1, b_e1, W_e2, b_e2, W_n1, b_n1, W_n2, b_n2,
           W_r, b_r, W_o, b_o):
    n, m = nbr_fea_idx.shape
    e = n * m
    nl = W_e1.shape[0]
    b, p = crystal_atom_idx.shape

    edge_attr_p = nbr_fea.reshape(e // 8, 128)
    dst = jnp.clip(nbr_fea_idx.reshape(e), 0, n - 1).astype(jnp.int32)
    zinit = jnp.zeros((n, 16), _F32)

    node, ps, pd, edge_p, mask_p = _tc_init(
        atom_fea, edge_attr_p, W_atom, b_atom.reshape(1, 64),
        _kron8(W_edge), jnp.tile(b_edge, 8).reshape(1, 128),
        _kron8(jnp.ones((16, 16), _F32)),
        W_e1[0, 16:80], W_e1[0, 80:144])

    cntP = _sc_scatter_add(mask_p.reshape(e, 16), dst, zinit, 2000)
    rinv = _tc_rinv(cntP)

    for l in range(nl):
        G = _sc_gather(pd, dst, 1000)
        edge_p, masked_p = _tc_edge(
            edge_p, G.reshape(e // 2, 128), ps, mask_p,
            _kron8(W_e1[l, :16]), jnp.tile(b_e1[l], 8).reshape(1, 512),
            _kron8(W_e2[l]), jnp.tile(b_e2[l], 8).reshape(1, 128))
        aggP = _sc_scatter_add(masked_p.reshape(e, 16), dst, zinit, 2000)
        ln = (l + 1) % nl
        node, ps, pd = _tc_node(node, aggP, rinv, W_n1[l, :64], W_n1[l, 64:80],
                                b_n1[l].reshape(1, 64), W_n2[l],
                                b_n2[l].reshape(1, 64), W_e1[ln, 16:80],
                                W_e1[ln, 80:144])

    ep = ((b * p + 255) // 256) * 256
    cai = jnp.concatenate([crystal_atom_idx.reshape(b * p).astype(jnp.int32),
                           jnp.zeros((ep - b * p,), jnp.int32)])
    R = _sc_gather(node, cai, ep // 32)
    col = jnp.arange(ep)
    avg = ((col[None, :] // p == jnp.arange(b)[:, None])
           & (col[None, :] < b * p)).astype(_F32) / p
    return _tc_readout(R, avg, W_r, b_r.reshape(1, 128), W_o,
                       b_o.reshape(1, 1))
